# baseline (speedup 1.0000x reference)
import jax
import jax.numpy as jnp
from jax import lax
from jax.experimental import pallas as pl
from jax.experimental.pallas import tpu as pltpu
from jax.experimental.pallas import tpu_sc as plsc

N = 10000
E = 320000
E2 = 160000
ENH = 80000
ET = E + ENH
IN_F = 128
OUT_F = 128
NREL = 16
ALPHA = 0.2

NC, NS, LANES = 2, 16, 16
NW = NC * NS
FPT = OUT_F // NW
CH = 4000
WCH = 2000
UNROLL = 5
NB = 1000

_SC_PARAMS = pltpu.CompilerParams(needs_layout_passes=False)


def _prep_body(x_ref, A1T, A2T, B1T, B2T, a2T, na2T,
               u_ref, v_ref, u2_ref, v2_ref, p_ref, q_ref, p2_ref, q2_ref):
    xb = x_ref[...]
    u = jnp.dot(xb, A1T[...], preferred_element_type=jnp.float32)
    v = jnp.dot(xb, A2T[...], preferred_element_type=jnp.float32)
    u2 = jnp.dot(xb, B1T[...], preferred_element_type=jnp.float32)
    v2 = jnp.dot(xb, B2T[...], preferred_element_type=jnp.float32)
    u_ref[...] = u
    v_ref[...] = v
    u2_ref[...] = u2
    v2_ref[...] = v2
    p_ref[...] = jnp.dot(u, a2T[...], preferred_element_type=jnp.float32)
    q_ref[...] = jnp.dot(v, a2T[...], preferred_element_type=jnp.float32)
    p2_ref[...] = jnp.dot(u2, na2T[...], preferred_element_type=jnp.float32)
    q2_ref[...] = jnp.dot(v2, na2T[...], preferred_element_type=jnp.float32)


def _node_prep(x, A1T, A2T, B1T, B2T, a2T, na2T):
    nblk = N // NB
    big = jax.ShapeDtypeStruct((N, OUT_F), jnp.float32)
    small = jax.ShapeDtypeStruct((N, 1), jnp.float32)
    wspec = pl.BlockSpec((IN_F, OUT_F), lambda i: (0, 0))
    vspec = pl.BlockSpec((IN_F, 1), lambda i: (0, 0))
    return pl.pallas_call(
        _prep_body,
        grid=(nblk,),
        in_specs=[pl.BlockSpec((NB, IN_F), lambda i: (i, 0)),
                  wspec, wspec, wspec, wspec, vspec, vspec],
        out_specs=[pl.BlockSpec((NB, OUT_F), lambda i: (i, 0))] * 4
                  + [pl.BlockSpec((NB, 1), lambda i: (i, 0))] * 4,
        out_shape=[big, big, big, big, small, small, small, small],
    )(x, A1T, A2T, B1T, B2T, a2T, na2T)


def _r1_body(eeT_ref, A3_ref, a2_ref, r_ref):
    w3 = jnp.dot(a2_ref[...], A3_ref[...], preferred_element_type=jnp.float32)
    r_ref[...] = jnp.dot(w3, eeT_ref[...], preferred_element_type=jnp.float32)


def _edge_r1(eeT, A3, a2):
    BE = 16000
    return pl.pallas_call(
        _r1_body,
        grid=(ET // BE,),
        in_specs=[pl.BlockSpec((NREL, BE), lambda i: (0, i)),
                  pl.BlockSpec((OUT_F, NREL), lambda i: (0, 0)),
                  pl.BlockSpec((1, OUT_F), lambda i: (0, 0))],
        out_specs=pl.BlockSpec((1, BE), lambda i: (0, i)),
        out_shape=jax.ShapeDtypeStruct((1, ET), jnp.float32),
    )(eeT, A3, a2)


def _r2_body(neeT_ref, oth_ref, B3_ref, B4_ref, na2_ref, r_ref):
    w3 = jnp.dot(na2_ref[...], B3_ref[...], preferred_element_type=jnp.float32)
    w4 = jnp.dot(na2_ref[...], B4_ref[...], preferred_element_type=jnp.float32)
    r_ref[...] = (jnp.dot(w3, neeT_ref[...], preferred_element_type=jnp.float32)
                  + jnp.dot(w4, oth_ref[...], preferred_element_type=jnp.float32))


def _edge_r2(neeT, otherM, B3, B4, na2):
    BE = 16000
    return pl.pallas_call(
        _r2_body,
        grid=(E2 // BE,),
        in_specs=[pl.BlockSpec((NREL, BE), lambda i: (0, i)),
                  pl.BlockSpec((4, BE), lambda i: (0, i)),
                  pl.BlockSpec((OUT_F, NREL), lambda i: (0, 0)),
                  pl.BlockSpec((OUT_F, 4), lambda i: (0, 0)),
                  pl.BlockSpec((1, OUT_F), lambda i: (0, 0))],
        out_specs=pl.BlockSpec((1, BE), lambda i: (0, i)),
        out_shape=jax.ShapeDtypeStruct((1, E2), jnp.float32),
    )(neeT, otherM, B3, B4, na2)


def _sc_weights_body(src1, dst1, r1, p1, q1, src2, dst2, r2, p2, q2,
                     w1_out, w2_out, p_v, q_v, sb, db, rb, wb):
    wid = lax.axis_index("s") * NC + lax.axis_index("c")

    def do_branch(nE, src_h, dst_h, r_h, p_h, q_h, w_out):
        nchunks = nE // WCH
        count = jnp.where(wid < nchunks % NW, nchunks // NW + 1, nchunks // NW)
        pltpu.sync_copy(p_h, p_v)
        pltpu.sync_copy(q_h, q_v)

        def chunk_body(k, _):
            base = (wid + NW * k) * WCH
            pltpu.sync_copy(src_h.at[pl.ds(base, WCH)], sb)
            pltpu.sync_copy(dst_h.at[pl.ds(base, WCH)], db)
            pltpu.sync_copy(r_h.at[pl.ds(base, WCH)], rb)

            def body(i, _):
                for k2 in range(UNROLL):
                    off = i * (UNROLL * LANES) + k2 * LANES
                    sidx = sb[pl.ds(off, LANES)]
                    didx = db[pl.ds(off, LANES)]
                    rv = rb[pl.ds(off, LANES)]
                    s = (plsc.load_gather(p_v, [sidx])
                         + plsc.load_gather(q_v, [didx]) + rv)
                    wb[pl.ds(off, LANES)] = jnp.exp(
                        -jnp.where(s > 0, s, ALPHA * s))
                return 0

            lax.fori_loop(0, WCH // (UNROLL * LANES), body, 0)
            pltpu.sync_copy(wb, w_out.at[pl.ds(base, WCH)])
            return 0

        lax.fori_loop(0, count, chunk_body, 0)

    do_branch(ET, src1, dst1, r1, p1, q1, w1_out)
    do_branch(E2, src2, dst2, r2, p2, q2, w2_out)


def _sc_weights(src1, dst1, r1, p1, q1, src2, dst2, r2, p2, q2):
    mesh = plsc.VectorSubcoreMesh(core_axis_name="c", subcore_axis_name="s",
                                  num_cores=NC, num_subcores=NS)
    out_type = (jax.ShapeDtypeStruct((ET,), jnp.float32),
                jax.ShapeDtypeStruct((E2,), jnp.float32))
    scratch = [
        pltpu.VMEM((N,), jnp.float32),
        pltpu.VMEM((N,), jnp.float32),
        pltpu.VMEM((WCH,), jnp.int32),
        pltpu.VMEM((WCH,), jnp.int32),
        pltpu.VMEM((WCH,), jnp.float32),
        pltpu.VMEM((WCH,), jnp.float32),
    ]
    return pl.kernel(
        _sc_weights_body,
        out_type=out_type,
        mesh=mesh,
        scratch_types=scratch,
        compiler_params=_SC_PARAMS,
    )(src1, dst1, r1, p1, q1, src2, dst2, r2, p2, q2)


def _zero_vmem(ref, nwords):
    z = jnp.zeros((LANES,), jnp.float32)

    def body(i, _):
        ref[pl.ds(i * LANES, LANES)] = z
        return 0

    lax.fori_loop(0, nwords // LANES, body, 0)


def _sc_branch(wid, nE, src_h, dst_h, w_h, extra_h, extra_base,
               vS_h, S2_out, t16_out, t4_out, rs_out, rs_row,
               n_t4, v_v, acc4, acc1, sbuf, dbuf, wbuf, ebuf, sems):
    has_ee = wid < 16
    is_rs = wid == 16
    has_t4 = jnp.logical_and(wid >= 17, wid < 17 + n_t4)
    has_extra = jnp.logical_or(has_ee, jnp.logical_or(is_rs, has_t4))
    has_stream = jnp.logical_and(has_extra, jnp.logical_not(is_rs))
    erow_off = jnp.where(has_ee, wid * nE,
                         jnp.where(has_t4, extra_base + (wid - 17) * nE, 0))

    pltpu.sync_copy(vS_h.at[wid], v_v)
    _zero_vmem(acc4, N * FPT)
    _zero_vmem(acc1, N)

    nchunks = nE // CH

    def _copies(c, slot):
        base = c * CH
        boff = slot * CH
        cps = [pltpu.make_async_copy(src_h.at[pl.ds(base, CH)],
                                     sbuf.at[pl.ds(boff, CH)], sems.at[0, slot]),
               pltpu.make_async_copy(dst_h.at[pl.ds(base, CH)],
                                     dbuf.at[pl.ds(boff, CH)], sems.at[1, slot]),
               pltpu.make_async_copy(w_h.at[pl.ds(base, CH)],
                                     wbuf.at[pl.ds(boff, CH)], sems.at[2, slot])]
        ecp = pltpu.make_async_copy(extra_h.at[pl.ds(erow_off + base, CH)],
                                    ebuf.at[pl.ds(boff, CH)], sems.at[3, slot])
        return cps, ecp

    def _start(c, slot):
        cps, ecp = _copies(c, slot)
        for cp in cps:
            cp.start()

        @pl.when(has_stream)
        def _():
            ecp.start()

    def _wait(c, slot):
        cps, ecp = _copies(c, slot)
        for cp in cps:
            cp.wait()

        @pl.when(has_stream)
        def _():
            ecp.wait()

    is_rs_b = lax.broadcast(is_rs, (LANES,))
    has_extra_b = lax.broadcast(has_extra, (LANES,))
    ones = jnp.ones((LANES,), jnp.float32)

    def _process(slot):
        boff = slot * CH

        def body(i, _):
            for k in range(UNROLL):
                off = boff + i * (UNROLL * LANES) + k * LANES
                sidx = sbuf[pl.ds(off, LANES)]
                didx = dbuf[pl.ds(off, LANES)]
                w = wbuf[pl.ds(off, LANES)]
                for f in range(FPT):
                    col = plsc.load_gather(v_v, [didx + f * N])
                    plsc.addupdate_scatter(acc4, [sidx + f * N], w * col)
                ev = ebuf[pl.ds(off, LANES)]
                em = jnp.where(is_rs_b, ones, ev)
                plsc.addupdate_scatter(acc1, [sidx], w * em, mask=has_extra_b)
            return 0

        lax.fori_loop(0, CH // (UNROLL * LANES), body, 0)

    _start(0, 0)

    def outer(c2, _):
        ca = 2 * c2
        _wait(ca, 0)
        _start(ca + 1, 1)
        _process(0)
        _wait(ca + 1, 1)

        @pl.when(ca + 2 < nchunks)
        def _():
            _start(ca + 2, 0)

        _process(1)
        return 0

    lax.fori_loop(0, nchunks // 2, outer, 0)

    pltpu.sync_copy(acc4, S2_out.at[wid])

    @pl.when(has_ee)
    def _():
        pltpu.sync_copy(acc1, t16_out.at[wid])

    @pl.when(is_rs)
    def _():
        pltpu.sync_copy(acc1, rs_out.at[rs_row])

    @pl.when(has_t4)
    def _():
        pltpu.sync_copy(acc1, t4_out.at[wid - 17])


def _sc_kernel_body(src1, dst1, w1, ee_s, vS1,
                    src2, dst2, w2, nee_s, vS2,
                    S2a, S2b, t16a, t16b, t4, rs,
                    v_v, acc4, acc1, sbuf, dbuf, wbuf, ebuf, sems):
    wid = lax.axis_index("s") * NC + lax.axis_index("c")
    _sc_branch(wid, ET, src1, dst1, w1, ee_s, 0,
               vS1, S2a, t16a, t4, rs, 0,
               0, v_v, acc4, acc1, sbuf, dbuf, wbuf, ebuf, sems)
    _sc_branch(wid, E2, src2, dst2, w2, nee_s, 16 * E2,
               vS2, S2b, t16b, t4, rs, 1,
               4, v_v, acc4, acc1, sbuf, dbuf, wbuf, ebuf, sems)


def _sc_aggregate(src1, dst1, w1, ee_s, vS1,
                  src2, dst2, w2, nee_s, vS2):
    mesh = plsc.VectorSubcoreMesh(core_axis_name="c", subcore_axis_name="s",
                                  num_cores=NC, num_subcores=NS)
    out_type = (
        jax.ShapeDtypeStruct((NW, N * FPT), jnp.float32),
        jax.ShapeDtypeStruct((NW, N * FPT), jnp.float32),
        jax.ShapeDtypeStruct((16, N), jnp.float32),
        jax.ShapeDtypeStruct((16, N), jnp.float32),
        jax.ShapeDtypeStruct((4, N), jnp.float32),
        jax.ShapeDtypeStruct((2, N), jnp.float32),
    )
    scratch = [
        pltpu.VMEM((N * FPT,), jnp.float32),
        pltpu.VMEM((N * FPT,), jnp.float32),
        pltpu.VMEM((N,), jnp.float32),
        pltpu.VMEM((2 * CH,), jnp.int32),
        pltpu.VMEM((2 * CH,), jnp.int32),
        pltpu.VMEM((2 * CH,), jnp.float32),
        pltpu.VMEM((2 * CH,), jnp.float32),
        pltpu.SemaphoreType.DMA((4, 2)),
    ]
    return pl.kernel(
        _sc_kernel_body,
        out_type=out_type,
        mesh=mesh,
        scratch_types=scratch,
        compiler_params=_SC_PARAMS,
    )(src1, dst1, w1, ee_s, vS1,
      src2, dst2, w2, nee_s, vS2)


def _final_body(u_ref, S2_ref, t16_ref, rs_ref, A3T_ref,
                u2_ref, S2b_ref, t16b_ref, t4_ref, rs2_ref, B3T_ref, B4T_ref,
                o1_ref, o2_ref):
    rs = rs_ref[...]
    h = (rs * u_ref[...] + S2_ref[...]
         + jnp.dot(t16_ref[...], A3T_ref[...], preferred_element_type=jnp.float32))
    h = h / jnp.where(rs == 0.0, 1e-12, rs)
    o1_ref[...] = jnp.where(h > 0, h, jnp.exp(h) - 1.0)
    rs2 = rs2_ref[...]
    h2 = (rs2 * u2_ref[...] + S2b_ref[...]
          + jnp.dot(t16b_ref[...], B3T_ref[...], preferred_element_type=jnp.float32)
          + jnp.dot(t4_ref[...], B4T_ref[...], preferred_element_type=jnp.float32))
    h2 = h2 / jnp.where(rs2 == 0.0, 1e-12, rs2)
    o2_ref[...] = jnp.where(h2 > 0, h2, jnp.exp(h2) - 1.0)


def _finalize(u, S2, t16, rs1, A3T, u2, S2b, t16b, t4, rs2, B3T, B4T):
    nblk = N // NB
    blk = lambda w: pl.BlockSpec((NB, w), lambda i: (i, 0))
    full = lambda s0, s1: pl.BlockSpec((s0, s1), lambda i: (0, 0))
    out = jax.ShapeDtypeStruct((N, OUT_F), jnp.float32)
    return pl.pallas_call(
        _final_body,
        grid=(nblk,),
        in_specs=[blk(OUT_F), blk(OUT_F), blk(NREL), blk(1), full(NREL, OUT_F),
                  blk(OUT_F), blk(OUT_F), blk(NREL), blk(4), blk(1),
                  full(NREL, OUT_F), full(4, OUT_F)],
        out_specs=[blk(OUT_F), blk(OUT_F)],
        out_shape=[out, out],
    )(u, S2, t16, rs1, A3T, u2, S2b, t16b, t4, rs2, B3T, B4T)


def kernel(input, edge, new_edge, edge_embed, new_edge_embed, new_edge_other,
           edge_list_nhop, edge_embed_nhop, a, new_a, a_2, new_a_2):
    x = input
    A1T = a[:, :IN_F].T
    A2T = a[:, IN_F:2 * IN_F].T
    A3 = a[:, 2 * IN_F:]
    B1T = new_a[:, :IN_F].T
    B2T = new_a[:, IN_F:2 * IN_F].T
    B3 = new_a[:, 2 * IN_F:2 * IN_F + NREL]
    B4 = new_a[:, 2 * IN_F + NREL:]
    a2T = a_2.T
    na2T = new_a_2.T

    src1 = jnp.concatenate([edge[0], edge_list_nhop[0]])
    dst1 = jnp.concatenate([edge[1], edge_list_nhop[1]])
    src2, dst2 = new_edge[0], new_edge[1]
    eeT = jnp.concatenate([edge_embed, edge_embed_nhop], axis=0).T
    neeT = new_edge_embed.T
    otherM = new_edge_other

    u, v, u2, v2, p1, q1, p2, q2 = _node_prep(x, A1T, A2T, B1T, B2T, a2T, na2T)

    r1 = _edge_r1(eeT, A3, a_2).reshape(ET)
    r2 = _edge_r2(neeT, otherM, B3, B4, new_a_2).reshape(E2)

    w1, w2 = _sc_weights(src1, dst1, r1, p1.reshape(N), q1.reshape(N),
                         src2, dst2, r2, p2.reshape(N), q2.reshape(N))

    vS1 = v.T.reshape(NW, FPT * N)
    vS2 = v2.T.reshape(NW, FPT * N)

    ee_s = eeT.reshape(16 * ET)
    nee_s = jnp.concatenate([neeT.reshape(16 * E2), otherM.reshape(4 * E2)])

    S2a, S2b, t16a, t16b, t4, rs = _sc_aggregate(
        src1, dst1, w1, ee_s, vS1,
        src2, dst2, w2, nee_s, vS2)

    S2a_r = S2a.reshape(OUT_F, N).T
    S2b_r = S2b.reshape(OUT_F, N).T
    t16a_r = t16a.T
    t16b_r = t16b.T
    t4_r = t4.T
    rs1 = rs[0].reshape(N, 1)
    rs2 = rs[1].reshape(N, 1)

    out1, out2 = _finalize(u, S2a_r, t16a_r, rs1, A3.T,
                           u2, S2b_r, t16b_r, t4_r, rs2, B3.T, B4.T)
    return (out1, out2)


# speedup vs baseline: 1.1748x; 1.1748x over previous
import jax
import jax.numpy as jnp
from jax import lax
from jax.experimental import pallas as pl
from jax.experimental.pallas import tpu as pltpu
from jax.experimental.pallas import tpu_sc as plsc

N = 10000
E = 320000
E2 = 160000
ENH = 80000
ET = E + ENH
IN_F = 128
OUT_F = 128
NREL = 16
ALPHA = 0.2

NC, NS, LANES = 2, 16, 16
NW = NC * NS
FPT = OUT_F // NW
CH = 4000
WCH = 2000
UNROLL = 5
NB = 1000

_SC_PARAMS = pltpu.CompilerParams(needs_layout_passes=False)


def _prep_body(x_ref, A1T, A2T, B1T, B2T, a2T, na2T,
               u_ref, v_ref, u2_ref, v2_ref, p_ref, q_ref, p2_ref, q2_ref):
    xb = x_ref[...]
    u = jnp.dot(xb, A1T[...], preferred_element_type=jnp.float32)
    v = jnp.dot(xb, A2T[...], preferred_element_type=jnp.float32)
    u2 = jnp.dot(xb, B1T[...], preferred_element_type=jnp.float32)
    v2 = jnp.dot(xb, B2T[...], preferred_element_type=jnp.float32)
    u_ref[...] = u
    v_ref[...] = v
    u2_ref[...] = u2
    v2_ref[...] = v2
    p_ref[...] = jnp.dot(u, a2T[...], preferred_element_type=jnp.float32)
    q_ref[...] = jnp.dot(v, a2T[...], preferred_element_type=jnp.float32)
    p2_ref[...] = jnp.dot(u2, na2T[...], preferred_element_type=jnp.float32)
    q2_ref[...] = jnp.dot(v2, na2T[...], preferred_element_type=jnp.float32)


def _node_prep(x, A1T, A2T, B1T, B2T, a2T, na2T):
    nblk = N // NB
    big = jax.ShapeDtypeStruct((N, OUT_F), jnp.float32)
    small = jax.ShapeDtypeStruct((N, 1), jnp.float32)
    wspec = pl.BlockSpec((IN_F, OUT_F), lambda i: (0, 0))
    vspec = pl.BlockSpec((IN_F, 1), lambda i: (0, 0))
    return pl.pallas_call(
        _prep_body,
        grid=(nblk,),
        in_specs=[pl.BlockSpec((NB, IN_F), lambda i: (i, 0)),
                  wspec, wspec, wspec, wspec, vspec, vspec],
        out_specs=[pl.BlockSpec((NB, OUT_F), lambda i: (i, 0))] * 4
                  + [pl.BlockSpec((NB, 1), lambda i: (i, 0))] * 4,
        out_shape=[big, big, big, big, small, small, small, small],
    )(x, A1T, A2T, B1T, B2T, a2T, na2T)


def _r1_body(eeT_ref, A3_ref, a2_ref, r_ref):
    w3 = jnp.dot(a2_ref[...], A3_ref[...], preferred_element_type=jnp.float32)
    r_ref[...] = jnp.dot(w3, eeT_ref[...], preferred_element_type=jnp.float32)


def _edge_r1(eeT, A3, a2):
    BE = 16000
    return pl.pallas_call(
        _r1_body,
        grid=(ET // BE,),
        in_specs=[pl.BlockSpec((NREL, BE), lambda i: (0, i)),
                  pl.BlockSpec((OUT_F, NREL), lambda i: (0, 0)),
                  pl.BlockSpec((1, OUT_F), lambda i: (0, 0))],
        out_specs=pl.BlockSpec((1, BE), lambda i: (0, i)),
        out_shape=jax.ShapeDtypeStruct((1, ET), jnp.float32),
    )(eeT, A3, a2)


def _r2_body(neeT_ref, oth_ref, B3_ref, B4_ref, na2_ref, r_ref):
    w3 = jnp.dot(na2_ref[...], B3_ref[...], preferred_element_type=jnp.float32)
    w4 = jnp.dot(na2_ref[...], B4_ref[...], preferred_element_type=jnp.float32)
    r_ref[...] = (jnp.dot(w3, neeT_ref[...], preferred_element_type=jnp.float32)
                  + jnp.dot(w4, oth_ref[...], preferred_element_type=jnp.float32))


def _edge_r2(neeT, otherM, B3, B4, na2):
    BE = 16000
    return pl.pallas_call(
        _r2_body,
        grid=(E2 // BE,),
        in_specs=[pl.BlockSpec((NREL, BE), lambda i: (0, i)),
                  pl.BlockSpec((4, BE), lambda i: (0, i)),
                  pl.BlockSpec((OUT_F, NREL), lambda i: (0, 0)),
                  pl.BlockSpec((OUT_F, 4), lambda i: (0, 0)),
                  pl.BlockSpec((1, OUT_F), lambda i: (0, 0))],
        out_specs=pl.BlockSpec((1, BE), lambda i: (0, i)),
        out_shape=jax.ShapeDtypeStruct((1, E2), jnp.float32),
    )(neeT, otherM, B3, B4, na2)


def _sc_weights_body(src1, dst1, r1, p1, q1, src2, dst2, r2, p2, q2,
                     w1_out, w2_out, p_v, q_v, sb, db, rb, wb):
    wid = lax.axis_index("s") * NC + lax.axis_index("c")

    def do_branch(nE, src_h, dst_h, r_h, p_h, q_h, w_out):
        nchunks = nE // WCH
        count = jnp.where(wid < nchunks % NW, nchunks // NW + 1, nchunks // NW)
        pltpu.sync_copy(p_h, p_v)
        pltpu.sync_copy(q_h, q_v)

        def chunk_body(k, _):
            base = (wid + NW * k) * WCH
            pltpu.sync_copy(src_h.at[pl.ds(base, WCH)], sb)
            pltpu.sync_copy(dst_h.at[pl.ds(base, WCH)], db)
            pltpu.sync_copy(r_h.at[pl.ds(base, WCH)], rb)

            def body(i, _):
                for k2 in range(UNROLL):
                    off = i * (UNROLL * LANES) + k2 * LANES
                    sidx = sb[pl.ds(off, LANES)]
                    didx = db[pl.ds(off, LANES)]
                    rv = rb[pl.ds(off, LANES)]
                    s = (plsc.load_gather(p_v, [sidx])
                         + plsc.load_gather(q_v, [didx]) + rv)
                    wb[pl.ds(off, LANES)] = jnp.exp(
                        -jnp.where(s > 0, s, ALPHA * s))
                return 0

            lax.fori_loop(0, WCH // (UNROLL * LANES), body, 0)
            pltpu.sync_copy(wb, w_out.at[pl.ds(base, WCH)])
            return 0

        lax.fori_loop(0, count, chunk_body, 0)

    do_branch(ET, src1, dst1, r1, p1, q1, w1_out)
    do_branch(E2, src2, dst2, r2, p2, q2, w2_out)


def _sc_weights(src1, dst1, r1, p1, q1, src2, dst2, r2, p2, q2):
    mesh = plsc.VectorSubcoreMesh(core_axis_name="c", subcore_axis_name="s",
                                  num_cores=NC, num_subcores=NS)
    out_type = (jax.ShapeDtypeStruct((ET,), jnp.float32),
                jax.ShapeDtypeStruct((E2,), jnp.float32))
    scratch = [
        pltpu.VMEM((N,), jnp.float32),
        pltpu.VMEM((N,), jnp.float32),
        pltpu.VMEM((WCH,), jnp.int32),
        pltpu.VMEM((WCH,), jnp.int32),
        pltpu.VMEM((WCH,), jnp.float32),
        pltpu.VMEM((WCH,), jnp.float32),
    ]
    return pl.kernel(
        _sc_weights_body,
        out_type=out_type,
        mesh=mesh,
        scratch_types=scratch,
        compiler_params=_SC_PARAMS,
    )(src1, dst1, r1, p1, q1, src2, dst2, r2, p2, q2)


def _zero_vmem(ref, nwords):
    z = jnp.zeros((LANES,), jnp.float32)

    def body(i, _):
        ref[pl.ds(i * LANES, LANES)] = z
        return 0

    lax.fori_loop(0, nwords // LANES, body, 0)


def _sc_branch(wid, nE, src_h, dst_h, w_h, extra_h, extra_base,
               vS_h, S2_out, t16_out, t4_out, rs_out, rs_row,
               n_t4, v_v, acc4, acc1, sbuf, dbuf, wbuf, ebuf, sems):
    has_ee = wid < 16
    is_rs = wid == 16
    has_t4 = jnp.logical_and(wid >= 17, wid < 17 + n_t4)
    has_extra = jnp.logical_or(has_ee, jnp.logical_or(is_rs, has_t4))
    has_stream = jnp.logical_and(has_extra, jnp.logical_not(is_rs))
    erow_off = jnp.where(has_ee, wid * nE,
                         jnp.where(has_t4, extra_base + (wid - 17) * nE, 0))

    pltpu.sync_copy(vS_h.at[wid], v_v)
    _zero_vmem(acc4, N * FPT)
    _zero_vmem(acc1, N)

    nchunks = nE // CH

    def _copies(c, slot):
        base = c * CH
        boff = slot * CH
        cps = [pltpu.make_async_copy(src_h.at[pl.ds(base, CH)],
                                     sbuf.at[pl.ds(boff, CH)], sems.at[0, slot]),
               pltpu.make_async_copy(dst_h.at[pl.ds(base, CH)],
                                     dbuf.at[pl.ds(boff, CH)], sems.at[1, slot]),
               pltpu.make_async_copy(w_h.at[pl.ds(base, CH)],
                                     wbuf.at[pl.ds(boff, CH)], sems.at[2, slot])]
        ecp = pltpu.make_async_copy(extra_h.at[pl.ds(erow_off + base, CH)],
                                    ebuf.at[pl.ds(boff, CH)], sems.at[3, slot])
        return cps, ecp

    def _start(c, slot):
        cps, ecp = _copies(c, slot)
        for cp in cps:
            cp.start()

        @pl.when(has_stream)
        def _():
            ecp.start()

    def _wait(c, slot):
        cps, ecp = _copies(c, slot)
        for cp in cps:
            cp.wait()

        @pl.when(has_stream)
        def _():
            ecp.wait()

    is_rs_b = lax.broadcast(is_rs, (LANES,))
    has_extra_b = lax.broadcast(has_extra, (LANES,))
    ones = jnp.ones((LANES,), jnp.float32)

    def _process(slot):
        boff = slot * CH

        def body(i, _):
            for k in range(UNROLL):
                off = boff + i * (UNROLL * LANES) + k * LANES
                sidx = sbuf[pl.ds(off, LANES)]
                didx = dbuf[pl.ds(off, LANES)]
                w = wbuf[pl.ds(off, LANES)]
                for fp in range(FPT // 2):
                    pw = plsc.load_gather(v_v, [didx + fp * N])
                    c0, c1 = plsc.unpack(plsc.bitcast(pw, jnp.bfloat16),
                                         format=plsc.PackFormat.INTERLEAVED)
                    plsc.addupdate_scatter(acc4, [sidx + (2 * fp) * N],
                                           w * c0)
                    plsc.addupdate_scatter(acc4, [sidx + (2 * fp + 1) * N],
                                           w * c1)
                ev = ebuf[pl.ds(off, LANES)]
                em = jnp.where(is_rs_b, ones, ev)
                plsc.addupdate_scatter(acc1, [sidx], w * em, mask=has_extra_b)
            return 0

        lax.fori_loop(0, CH // (UNROLL * LANES), body, 0)

    _start(0, 0)

    def outer(c2, _):
        ca = 2 * c2
        _wait(ca, 0)
        _start(ca + 1, 1)
        _process(0)
        _wait(ca + 1, 1)

        @pl.when(ca + 2 < nchunks)
        def _():
            _start(ca + 2, 0)

        _process(1)
        return 0

    lax.fori_loop(0, nchunks // 2, outer, 0)

    pltpu.sync_copy(acc4, S2_out.at[wid])

    @pl.when(has_ee)
    def _():
        pltpu.sync_copy(acc1, t16_out.at[wid])

    @pl.when(is_rs)
    def _():
        pltpu.sync_copy(acc1, rs_out.at[rs_row])

    @pl.when(has_t4)
    def _():
        pltpu.sync_copy(acc1, t4_out.at[wid - 17])


def _sc_kernel_body(src1, dst1, w1, ee_s, vS1,
                    src2, dst2, w2, nee_s, vS2,
                    S2a, S2b, t16a, t16b, t4, rs,
                    v_v, acc4, acc1, sbuf, dbuf, wbuf, ebuf, sems):
    wid = lax.axis_index("s") * NC + lax.axis_index("c")
    _sc_branch(wid, ET, src1, dst1, w1, ee_s, 0,
               vS1, S2a, t16a, t4, rs, 0,
               0, v_v, acc4, acc1, sbuf, dbuf, wbuf, ebuf, sems)
    _sc_branch(wid, E2, src2, dst2, w2, nee_s, 16 * E2,
               vS2, S2b, t16b, t4, rs, 1,
               4, v_v, acc4, acc1, sbuf, dbuf, wbuf, ebuf, sems)


def _sc_aggregate(src1, dst1, w1, ee_s, vS1,
                  src2, dst2, w2, nee_s, vS2):
    mesh = plsc.VectorSubcoreMesh(core_axis_name="c", subcore_axis_name="s",
                                  num_cores=NC, num_subcores=NS)
    out_type = (
        jax.ShapeDtypeStruct((NW, N * FPT), jnp.float32),
        jax.ShapeDtypeStruct((NW, N * FPT), jnp.float32),
        jax.ShapeDtypeStruct((16, N), jnp.float32),
        jax.ShapeDtypeStruct((16, N), jnp.float32),
        jax.ShapeDtypeStruct((4, N), jnp.float32),
        jax.ShapeDtypeStruct((2, N), jnp.float32),
    )
    scratch = [
        pltpu.VMEM((N * FPT // 2,), jnp.int32),
        pltpu.VMEM((N * FPT,), jnp.float32),
        pltpu.VMEM((N,), jnp.float32),
        pltpu.VMEM((2 * CH,), jnp.int32),
        pltpu.VMEM((2 * CH,), jnp.int32),
        pltpu.VMEM((2 * CH,), jnp.float32),
        pltpu.VMEM((2 * CH,), jnp.float32),
        pltpu.SemaphoreType.DMA((4, 2)),
    ]
    return pl.kernel(
        _sc_kernel_body,
        out_type=out_type,
        mesh=mesh,
        scratch_types=scratch,
        compiler_params=_SC_PARAMS,
    )(src1, dst1, w1, ee_s, vS1,
      src2, dst2, w2, nee_s, vS2)


def _final_body(u_ref, S2_ref, t16_ref, rs_ref, A3T_ref,
                u2_ref, S2b_ref, t16b_ref, t4_ref, rs2_ref, B3T_ref, B4T_ref,
                o1_ref, o2_ref):
    rs = rs_ref[...]
    h = (rs * u_ref[...] + S2_ref[...]
         + jnp.dot(t16_ref[...], A3T_ref[...], preferred_element_type=jnp.float32))
    h = h / jnp.where(rs == 0.0, 1e-12, rs)
    o1_ref[...] = jnp.where(h > 0, h, jnp.exp(h) - 1.0)
    rs2 = rs2_ref[...]
    h2 = (rs2 * u2_ref[...] + S2b_ref[...]
          + jnp.dot(t16b_ref[...], B3T_ref[...], preferred_element_type=jnp.float32)
          + jnp.dot(t4_ref[...], B4T_ref[...], preferred_element_type=jnp.float32))
    h2 = h2 / jnp.where(rs2 == 0.0, 1e-12, rs2)
    o2_ref[...] = jnp.where(h2 > 0, h2, jnp.exp(h2) - 1.0)


def _finalize(u, S2, t16, rs1, A3T, u2, S2b, t16b, t4, rs2, B3T, B4T):
    nblk = N // NB
    blk = lambda w: pl.BlockSpec((NB, w), lambda i: (i, 0))
    full = lambda s0, s1: pl.BlockSpec((s0, s1), lambda i: (0, 0))
    out = jax.ShapeDtypeStruct((N, OUT_F), jnp.float32)
    return pl.pallas_call(
        _final_body,
        grid=(nblk,),
        in_specs=[blk(OUT_F), blk(OUT_F), blk(NREL), blk(1), full(NREL, OUT_F),
                  blk(OUT_F), blk(OUT_F), blk(NREL), blk(4), blk(1),
                  full(NREL, OUT_F), full(4, OUT_F)],
        out_specs=[blk(OUT_F), blk(OUT_F)],
        out_shape=[out, out],
    )(u, S2, t16, rs1, A3T, u2, S2b, t16b, t4, rs2, B3T, B4T)


def kernel(input, edge, new_edge, edge_embed, new_edge_embed, new_edge_other,
           edge_list_nhop, edge_embed_nhop, a, new_a, a_2, new_a_2):
    x = input
    A1T = a[:, :IN_F].T
    A2T = a[:, IN_F:2 * IN_F].T
    A3 = a[:, 2 * IN_F:]
    B1T = new_a[:, :IN_F].T
    B2T = new_a[:, IN_F:2 * IN_F].T
    B3 = new_a[:, 2 * IN_F:2 * IN_F + NREL]
    B4 = new_a[:, 2 * IN_F + NREL:]
    a2T = a_2.T
    na2T = new_a_2.T

    src1 = jnp.concatenate([edge[0], edge_list_nhop[0]])
    dst1 = jnp.concatenate([edge[1], edge_list_nhop[1]])
    src2, dst2 = new_edge[0], new_edge[1]
    eeT = jnp.concatenate([edge_embed, edge_embed_nhop], axis=0).T
    neeT = new_edge_embed.T
    otherM = new_edge_other

    u, v, u2, v2, p1, q1, p2, q2 = _node_prep(x, A1T, A2T, B1T, B2T, a2T, na2T)

    r1 = _edge_r1(eeT, A3, a_2).reshape(ET)
    r2 = _edge_r2(neeT, otherM, B3, B4, new_a_2).reshape(E2)

    w1, w2 = _sc_weights(src1, dst1, r1, p1.reshape(N), q1.reshape(N),
                         src2, dst2, r2, p2.reshape(N), q2.reshape(N))

    def _pack_pairs(vm):
        vt = vm.T.astype(jnp.bfloat16)
        vpair = jnp.swapaxes(vt.reshape(OUT_F // 2, 2, N), 1, 2)
        vwords = jax.lax.bitcast_convert_type(vpair, jnp.int32)
        return vwords.reshape(NW, (FPT // 2) * N)

    vS1 = _pack_pairs(v)
    vS2 = _pack_pairs(v2)

    ee_s = eeT.reshape(16 * ET)
    nee_s = jnp.concatenate([neeT.reshape(16 * E2), otherM.reshape(4 * E2)])

    S2a, S2b, t16a, t16b, t4, rs = _sc_aggregate(
        src1, dst1, w1, ee_s, vS1,
        src2, dst2, w2, nee_s, vS2)

    S2a_r = S2a.reshape(OUT_F, N).T
    S2b_r = S2b.reshape(OUT_F, N).T
    t16a_r = t16a.T
    t16b_r = t16b.T
    t4_r = t4.T
    rs1 = rs[0].reshape(N, 1)
    rs2 = rs[1].reshape(N, 1)

    out1, out2 = _finalize(u, S2a_r, t16a_r, rs1, A3.T,
                           u2, S2b_r, t16b_r, t4_r, rs2, B3.T, B4.T)
    return (out1, out2)
